# baseline (speedup 1.0000x reference)
import jax
import jax.numpy as jnp
from jax import lax
from jax.experimental import pallas as pl
from jax.experimental.pallas import tpu as pltpu
from jax.experimental.pallas import tpu_sc as plsc

_N = 10000
_E = 320000
_D = 128
_H = 64
_O = 4
_R = 3

_NPAD = 10240
_SN = _R * _NPAD
_SNA = _SN + 128
_K = 128
_ER = _E // _K
_ERP = 2560
_EPAD = _ERP * _K
_STRIPE = _SNA // 16

_NCH1 = _ERP // 16
_NCH2 = _ERP // 32

_BLK = 1024
_NB = _NPAD // _BLK
_HW = _H // 2

_mesh = plsc.VectorSubcoreMesh(core_axis_name="c", subcore_axis_name="s")
_sc_params = pltpu.CompilerParams(use_tc_tiling_on_sc=False)


def _mm_body(x_ref, w_ref, b_ref, r1_ref, ys_ref):
    y = (
        jnp.dot(x_ref[...], w_ref[...], preferred_element_type=jnp.float32)
        + b_ref[...]
    )
    r1_ref[...] = y[:, _H:]
    ys_ref[0] = y[:, :_HW]
    ys_ref[1] = y[:, _HW:_H]


def _edge_prep_body(ei_ref, t_ref, src_ref, sidx_ref):
    i = pl.program_id(0)
    rows = _ERP // 10
    row0 = i * rows
    rid = jax.lax.broadcasted_iota(jnp.int32, (rows, _K), 0) + row0
    live = rid < _ER
    src_ref[...] = jnp.where(live, ei_ref[0], 0)
    sidx_ref[...] = jnp.where(live, ei_ref[1] + t_ref[...] * _NPAD, _SN)


def _seg1_body(src_hbm, sidx_hbm, y_hbm, za_hbm, zc_hbm, ones_hbm,
               aout_hbm, cout_hbm,
               a_sh, c_sh, src_v, sidx_v, rows0_v, rows1_v, ones_v,
               sem0, sem1):
    cid = lax.axis_index("c")
    sid = lax.axis_index("s")
    base = sid * _STRIPE

    pltpu.sync_copy(za_hbm, a_sh.at[pl.ds(base, _STRIPE)])
    pltpu.sync_copy(zc_hbm, c_sh.at[pl.ds(base, _STRIPE)])
    pltpu.sync_copy(ones_hbm, ones_v)

    pltpu.sync_copy(src_hbm.at[sid], src_v)
    pltpu.sync_copy(sidx_hbm.at[sid], sidx_v)
    plsc.subcore_barrier()

    do_counts = cid == 0
    ysrc = y_hbm.at[cid]

    def gather(ci, buf, sem):
        pltpu.async_copy(ysrc.at[src_v.at[ci]], buf, sem)

    def drain_scatter(ci, buf, sem):
        pltpu.make_async_copy(ysrc.at[src_v.at[ci]], buf, sem).wait()
        pltpu.sync_copy(buf, a_sh.at[sidx_v.at[ci]], add=True)

        @pl.when(do_counts)
        def _():
            pltpu.sync_copy(ones_v, c_sh.at[sidx_v.at[ci]], add=True)

    gather(0, rows0_v, sem0)

    def pair(g, _):
        c0 = 2 * g
        gather(c0 + 1, rows1_v, sem1)
        drain_scatter(c0, rows0_v, sem0)

        @pl.when(c0 + 2 < _NCH1)
        def _():
            gather(c0 + 2, rows0_v, sem0)
        drain_scatter(c0 + 1, rows1_v, sem1)
        return 0
    lax.fori_loop(0, _NCH1 // 2, pair, 0)

    plsc.subcore_barrier()
    pltpu.sync_copy(a_sh.at[pl.ds(base, _STRIPE)],
                    aout_hbm.at[cid, pl.ds(base, _STRIPE)])

    @pl.when(do_counts)
    def _():
        pltpu.sync_copy(c_sh.at[pl.ds(base, _STRIPE)],
                        cout_hbm.at[pl.ds(base, _STRIPE)])


def _seg2_body(src_hbm, sidx_hbm, y_hbm, za_hbm, aout_hbm,
               a_sh, src_v, sidx_v, rows0_v, rows1_v, sem0, sem1):
    cid = lax.axis_index("c")
    sid = lax.axis_index("s")
    wid = sid * 2 + cid
    base = sid * _STRIPE

    pltpu.sync_copy(za_hbm, a_sh.at[pl.ds(base, _STRIPE)])
    pltpu.sync_copy(src_hbm.at[wid], src_v)
    pltpu.sync_copy(sidx_hbm.at[wid], sidx_v)
    plsc.subcore_barrier()

    def gather(ci, buf, sem):
        pltpu.async_copy(y_hbm.at[src_v.at[ci]], buf, sem)

    def drain_scatter(ci, buf, sem):
        pltpu.make_async_copy(y_hbm.at[src_v.at[ci]], buf, sem).wait()
        pltpu.sync_copy(buf, a_sh.at[sidx_v.at[ci]], add=True)

    gather(0, rows0_v, sem0)

    def pair(g, _):
        c0 = 2 * g
        gather(c0 + 1, rows1_v, sem1)
        drain_scatter(c0, rows0_v, sem0)

        @pl.when(c0 + 2 < _NCH2)
        def _():
            gather(c0 + 2, rows0_v, sem0)
        drain_scatter(c0 + 1, rows1_v, sem1)
        return 0
    lax.fori_loop(0, _NCH2 // 2, pair, 0)

    plsc.subcore_barrier()
    pltpu.sync_copy(a_sh.at[pl.ds(base, _STRIPE)],
                    aout_hbm.at[cid, pl.ds(base, _STRIPE)])


_seg1 = pl.kernel(
    _seg1_body,
    out_type=(
        jax.ShapeDtypeStruct((2, _SNA, _HW), jnp.float32),
        jax.ShapeDtypeStruct((_SNA, 1), jnp.float32),
    ),
    mesh=_mesh,
    scratch_types=[
        pltpu.VMEM_SHARED((_SNA, _HW), jnp.float32),
        pltpu.VMEM_SHARED((_SNA, 1), jnp.float32),
        pltpu.VMEM((_NCH1, _K), jnp.int32),
        pltpu.VMEM((_NCH1, _K), jnp.int32),
        pltpu.VMEM((_K, _HW), jnp.float32),
        pltpu.VMEM((_K, _HW), jnp.float32),
        pltpu.VMEM((_K, 1), jnp.float32),
        pltpu.SemaphoreType.DMA,
        pltpu.SemaphoreType.DMA,
    ],
    compiler_params=_sc_params,
)

_seg2 = pl.kernel(
    _seg2_body,
    out_type=jax.ShapeDtypeStruct((2, _SNA, 2 * _O), jnp.float32),
    mesh=_mesh,
    scratch_types=[
        pltpu.VMEM_SHARED((_SNA, 2 * _O), jnp.float32),
        pltpu.VMEM((_NCH2, _K), jnp.int32),
        pltpu.VMEM((_NCH2, _K), jnp.int32),
        pltpu.VMEM((_K, 2 * _O), jnp.float32),
        pltpu.VMEM((_K, 2 * _O), jnp.float32),
        pltpu.SemaphoreType.DMA,
        pltpu.SemaphoreType.DMA,
    ],
    compiler_params=_sc_params,
)


def _comb1_body(r1_ref, a0_ref, a1_ref, a2_ref, c0_ref, c1_ref, c2_ref,
                comp_ref, w_ref, b_ref, o_ref):
    acc = r1_ref[...]
    for r, (a_ref, c_ref) in enumerate(
            ((a0_ref, c0_ref), (a1_ref, c1_ref), (a2_ref, c2_ref))):
        s = jnp.concatenate([a_ref[0], a_ref[1]], axis=1)
        invc = comp_ref[r, 0] / jnp.maximum(c_ref[...], 1.0)
        acc = acc + s * invc
    h = jnp.maximum(acc, 0.0)
    o_ref[...] = (
        jnp.dot(h, w_ref[...], preferred_element_type=jnp.float32) + b_ref[...]
    )


def _comb2_body(y_ref, a0_ref, a1_ref, a2_ref, c0_ref, c1_ref, c2_ref,
                comp_ref, o_ref):
    acc = y_ref[:, _O:]
    for r, (a_ref, c_ref) in enumerate(
            ((a0_ref, c0_ref), (a1_ref, c1_ref), (a2_ref, c2_ref))):
        s = a_ref[0, :, :_O] + a_ref[1, :, :_O]
        invc = comp_ref[r, 0] / jnp.maximum(c_ref[...], 1.0)
        acc = acc + s * invc
    z = acc - jnp.max(acc, axis=1, keepdims=True)
    ez = jnp.exp(z)
    o_ref[...] = ez / jnp.sum(ez, axis=1, keepdims=True)


def _a_spec(width, r):
    return pl.BlockSpec((2, _BLK, width), lambda i, r=r: (0, r * _NB + i, 0))


def _c_spec(r):
    return pl.BlockSpec((_BLK, 1), lambda i, r=r: (r * _NB + i, 0))


def kernel(x, edge_index, edge_type, basis1, comp1, root1, bias1,
           basis2, comp2, root2, bias2):
    ei3 = edge_index.astype(jnp.int32).reshape(2, _ER, _K)
    t2 = edge_type.astype(jnp.int32).reshape(_ER, _K)

    erows = _ERP // 10
    src_p, sidx_p = pl.pallas_call(
        _edge_prep_body,
        grid=(10,),
        in_specs=[
            pl.BlockSpec((2, erows, _K), lambda i: (0, i, 0)),
            pl.BlockSpec((erows, _K), lambda i: (i, 0)),
        ],
        out_specs=[
            pl.BlockSpec((erows, _K), lambda i: (i, 0)),
            pl.BlockSpec((erows, _K), lambda i: (i, 0)),
        ],
        out_shape=[
            jax.ShapeDtypeStruct((_ERP, _K), jnp.int32),
            jax.ShapeDtypeStruct((_ERP, _K), jnp.int32),
        ],
    )(ei3, t2)

    src16 = src_p.reshape(16, _NCH1, _K)
    sidx16 = sidx_p.reshape(16, _NCH1, _K)
    src32 = src_p.reshape(32, _NCH2, _K)
    sidx32 = sidx_p.reshape(32, _NCH2, _K)

    w1cat = jnp.concatenate([basis1[0], root1], axis=1)
    b1cat = jnp.concatenate([jnp.zeros((_H,), jnp.float32), bias1])[None, :]
    r1, y1s = pl.pallas_call(
        _mm_body,
        grid=(_N // 1000,),
        in_specs=[
            pl.BlockSpec((1000, _D), lambda i: (i, 0)),
            pl.BlockSpec((_D, 2 * _H), lambda i: (0, 0)),
            pl.BlockSpec((1, 2 * _H), lambda i: (0, 0)),
        ],
        out_specs=[
            pl.BlockSpec((1000, _H), lambda i: (i, 0)),
            pl.BlockSpec((2, 1000, _HW), lambda i: (0, i, 0)),
        ],
        out_shape=[
            jax.ShapeDtypeStruct((_N, _H), jnp.float32),
            jax.ShapeDtypeStruct((2, _N, _HW), jnp.float32),
        ],
    )(x, w1cat, b1cat)

    za1 = jnp.zeros((_STRIPE, _HW), jnp.float32)
    zc = jnp.zeros((_STRIPE, 1), jnp.float32)
    ones_k = jnp.ones((_K, 1), jnp.float32)
    a1p, c1 = _seg1(src16, sidx16, y1s, za1, zc, ones_k)

    w2cat = jnp.concatenate([basis2[0], root2], axis=1)
    b2cat = jnp.concatenate([jnp.zeros((_O,), jnp.float32), bias2])[None, :]
    y2r = pl.pallas_call(
        _comb1_body,
        grid=(_NB,),
        in_specs=[
            pl.BlockSpec((_BLK, _H), lambda i: (i, 0)),
            _a_spec(_HW, 0), _a_spec(_HW, 1), _a_spec(_HW, 2),
            _c_spec(0), _c_spec(1), _c_spec(2),
            pl.BlockSpec((_R, 1), lambda i: (0, 0)),
            pl.BlockSpec((_H, 2 * _O), lambda i: (0, 0)),
            pl.BlockSpec((1, 2 * _O), lambda i: (0, 0)),
        ],
        out_specs=pl.BlockSpec((_BLK, 2 * _O), lambda i: (i, 0)),
        out_shape=jax.ShapeDtypeStruct((_N, 2 * _O), jnp.float32),
    )(r1, a1p, a1p, a1p, c1, c1, c1, comp1, w2cat, b2cat)

    za2 = jnp.zeros((_STRIPE, 2 * _O), jnp.float32)
    a2p = _seg2(src32, sidx32, y2r, za2)

    out = pl.pallas_call(
        _comb2_body,
        grid=(_NB,),
        in_specs=[
            pl.BlockSpec((_BLK, 2 * _O), lambda i: (i, 0)),
            _a_spec(2 * _O, 0), _a_spec(2 * _O, 1), _a_spec(2 * _O, 2),
            _c_spec(0), _c_spec(1), _c_spec(2),
            pl.BlockSpec((_R, 1), lambda i: (0, 0)),
        ],
        out_specs=pl.BlockSpec((_BLK, _O), lambda i: (i, 0)),
        out_shape=jax.ShapeDtypeStruct((_N, _O), jnp.float32),
    )(y2r, a2p, a2p, a2p, c1, c1, c1, comp2)

    return out
